# Initial kernel scaffold; baseline (speedup 1.0000x reference)
import functools

import jax
import jax.numpy as jnp
from jax import lax
from jax.experimental import pallas as pl
from jax.experimental.pallas import tpu as pltpu
from jax.experimental.pallas import tpu_sc as plsc

_NC = 2
_NS = 16
_CHUNK = 80
_ZROWS = 125


def _evolve_matmul_body(f_ref, q_ref, uW, uU, ub, rW, rU, rb, hW, hU, hb,
                        o_ref, q3_ref):
    @pl.when(pl.program_id(0) == 0)
    def _():
        hp = jax.lax.Precision.HIGHEST
        dot = functools.partial(jnp.dot, precision=hp,
                                preferred_element_type=jnp.float32)
        Q = q_ref[...]
        for _ in range(3):
            z = Q
            upd = jax.nn.sigmoid(dot(uW[...], z) + dot(uU[...], Q) + ub[...])
            rst = jax.nn.sigmoid(dot(rW[...], z) + dot(rU[...], Q) + rb[...])
            hcap = jnp.tanh(dot(hW[...], z) + dot(hU[...], rst * Q) + hb[...])
            Q = (1.0 - upd) * Q + upd * hcap
        q3_ref[...] = Q

    o_ref[...] = jnp.dot(f_ref[...], q3_ref[...],
                         preferred_element_type=jnp.float32)


def _decoder_body(p_ref, w1, b1, w2, b2, o_ref):
    emb = jnp.maximum(p_ref[0] + p_ref[1], 0.0)
    h = jnp.maximum(
        jnp.dot(emb, w1[...], preferred_element_type=jnp.float32) + b1[...],
        0.0)
    o_ref[...] = jnp.dot(h, w2[...], preferred_element_type=jnp.float32) \
        + b2[...]


def _sc_scatter_body(n_nodes, n_chunks,
                     xq_hbm, src_hbm, dst_hbm, w_hbm, out_hbm,
                     src_v, dst_v, w_v, rows_v, zbuf, acc, sem):
    cid = lax.axis_index("c")
    sid = lax.axis_index("s")
    rows_per_tile = n_nodes // _NS

    pltpu.sync_copy(src_hbm.at[cid, sid], src_v)
    pltpu.sync_copy(dst_hbm.at[cid, sid], dst_v)
    pltpu.sync_copy(w_hbm.at[cid, sid], w_v)

    def _zrow(r, carry):
        for j in range(8):
            zbuf[r, pl.ds(j * 16, 16)] = jnp.zeros((16,), jnp.float32)
        return carry
    lax.fori_loop(0, _ZROWS, _zrow, 0)
    base = sid * rows_per_tile
    for t in range(rows_per_tile // _ZROWS):
        pltpu.sync_copy(zbuf, acc.at[pl.ds(base + t * _ZROWS, _ZROWS)])
    plsc.subcore_barrier()

    def _chunk(k, carry):
        pltpu.async_copy(xq_hbm.at[src_v.at[k]], rows_v, sem).wait()

        def _edge(e, c2):
            ws = w_v[k, e]
            for j in range(8):
                sl = pl.ds(j * 16, 16)
                rows_v[e, sl] = rows_v[e, sl] * ws
            return c2
        lax.fori_loop(0, _CHUNK, _edge, 0)
        pltpu.sync_copy(rows_v, acc.at[dst_v.at[k]], add=True)
        return carry
    lax.fori_loop(0, n_chunks, _chunk, 0)

    plsc.subcore_barrier()
    pltpu.sync_copy(acc.at[pl.ds(base, rows_per_tile)],
                    out_hbm.at[cid, pl.ds(base, rows_per_tile)])


def _sc_scatter(xq, src, dst, w, n_nodes, n_chunks, interpret=False):
    mesh = plsc.VectorSubcoreMesh(core_axis_name="c", subcore_axis_name="s")
    kern = pl.kernel(
        functools.partial(_sc_scatter_body, n_nodes, n_chunks),
        out_type=jax.ShapeDtypeStruct((_NC, n_nodes, 128), jnp.float32),
        mesh=mesh,
        scratch_types=[
            pltpu.VMEM((n_chunks, _CHUNK), jnp.int32),
            pltpu.VMEM((n_chunks, _CHUNK), jnp.int32),
            pltpu.VMEM((n_chunks, _CHUNK), jnp.float32),
            pltpu.VMEM((_CHUNK, 128), jnp.float32),
            pltpu.VMEM((_ZROWS, 128), jnp.float32),
            pltpu.VMEM_SHARED((n_nodes, 128), jnp.float32),
            pltpu.SemaphoreType.DMA,
        ],
        interpret=interpret,
    )
    return kern(xq, src, dst, w)


def kernel(edge_index, edge_weight, feats, GCN_init_weights,
           update_W, update_U, update_b,
           reset_W, reset_U, reset_b,
           htilda_W, htilda_U, htilda_b,
           dec_W1, dec_b1, dec_W2, dec_b2, pred_flag=1):
    T, n, d = feats.shape
    e = edge_weight.shape[-1]
    n_chunks = e // (_NC * _NS * _CHUNK)
    assert e == _NC * _NS * n_chunks * _CHUNK and n % _NS == 0 and d == 128

    feats2 = feats[T - 1]
    ei = edge_index[T - 1].astype(jnp.int32)
    shp = (_NC, _NS, n_chunks, _CHUNK)
    dst = ei[0].reshape(shp)
    src = ei[1].reshape(shp)
    w = edge_weight[T - 1].reshape(shp)

    blk = 1000
    full = pl.BlockSpec((128, 128), lambda i: (0, 0))
    xq = pl.pallas_call(
        _evolve_matmul_body,
        grid=(n // blk,),
        in_specs=[pl.BlockSpec((blk, 128), lambda i: (i, 0))] + [full] * 10,
        out_specs=pl.BlockSpec((blk, 128), lambda i: (i, 0)),
        out_shape=jax.ShapeDtypeStruct((n, 128), jnp.float32),
        scratch_shapes=[pltpu.VMEM((128, 128), jnp.float32)],
    )(feats2, GCN_init_weights, update_W, update_U, update_b,
      reset_W, reset_U, reset_b, htilda_W, htilda_U, htilda_b)

    partial = _sc_scatter(xq, src, dst, w, n, n_chunks)

    out = pl.pallas_call(
        _decoder_body,
        grid=(n // blk,),
        in_specs=[pl.BlockSpec((_NC, blk, 128), lambda i: (0, i, 0)),
                  pl.BlockSpec((128, 128), lambda i: (0, 0)),
                  pl.BlockSpec((1, 128), lambda i: (0, 0)),
                  pl.BlockSpec((128, 64), lambda i: (0, 0)),
                  pl.BlockSpec((1, 64), lambda i: (0, 0))],
        out_specs=pl.BlockSpec((blk, 64), lambda i: (i, 0)),
        out_shape=jax.ShapeDtypeStruct((n, 64), jnp.float32),
    )(partial, dec_W1, dec_b1.reshape(1, -1), dec_W2, dec_b2.reshape(1, -1))
    return out


# speedup vs baseline: 4.1765x; 4.1765x over previous
import functools

import jax
import jax.numpy as jnp
from jax import lax
from jax.experimental import pallas as pl
from jax.experimental.pallas import tpu as pltpu
from jax.experimental.pallas import tpu_sc as plsc

_NC = 2
_NS = 16
_CHUNK = 80


def _evolve_matmul_body(f_ref, q_ref, uW, uU, ub, rW, rU, rb, hW, hU, hb,
                        o_ref, q3_ref):
    @pl.when(pl.program_id(0) == 0)
    def _():
        hp = jax.lax.Precision.HIGHEST
        dot = functools.partial(jnp.dot, precision=hp,
                                preferred_element_type=jnp.float32)
        Q = q_ref[...]
        for _ in range(3):
            z = Q
            upd = jax.nn.sigmoid(dot(uW[...], z) + dot(uU[...], Q) + ub[...])
            rst = jax.nn.sigmoid(dot(rW[...], z) + dot(rU[...], Q) + rb[...])
            hcap = jnp.tanh(dot(hW[...], z) + dot(hU[...], rst * Q) + hb[...])
            Q = (1.0 - upd) * Q + upd * hcap
        q3_ref[...] = Q

    o_ref[...] = jnp.dot(f_ref[...], q3_ref[...],
                         preferred_element_type=jnp.float32)


def _decoder_body(p_ref, w1, b1, w2, b2, o_ref):
    emb = jnp.maximum(p_ref[0] + p_ref[1], 0.0)
    h = jnp.maximum(
        jnp.dot(emb, w1[...], preferred_element_type=jnp.float32) + b1[...],
        0.0)
    o_ref[...] = jnp.dot(h, w2[...], preferred_element_type=jnp.float32) \
        + b2[...]


def _sc_scatter_body(n_nodes, n_chunks,
                     xq_hbm, edges_hbm, w_hbm, out_hbm,
                     ibuf, wbuf, rows_v, acc, sem):
    cid = lax.axis_index("c")
    sid = lax.axis_index("s")
    n_own = 10
    rows_per_owner = n_nodes // n_own
    base = sid * rows_per_owner

    @pl.when(sid < n_own)
    def _():
        def _zrow(r, carry):
            for j in range(8):
                rows_v[r, pl.ds(j * 16, 16)] = jnp.zeros((16,), jnp.float32)
            return carry
        lax.fori_loop(0, _CHUNK, _zrow, 0)
        nfull, rem = divmod(rows_per_owner, _CHUNK)
        for t in range(nfull):
            pltpu.sync_copy(rows_v, acc.at[pl.ds(base + t * _CHUNK, _CHUNK)])
        if rem:
            pltpu.sync_copy(rows_v.at[pl.ds(0, rem)],
                            acc.at[pl.ds(base + nfull * _CHUNK, rem)])
    plsc.subcore_barrier()

    def _chunk(k, carry):
        pltpu.sync_copy(edges_hbm.at[cid, sid, k], ibuf)
        pltpu.sync_copy(w_hbm.at[cid, sid, k], wbuf)
        pltpu.async_copy(xq_hbm.at[ibuf.at[0]], rows_v, sem).wait()

        def _group(g, c2):
            wvec = wbuf[pl.ds(g * 16, 16)]
            for e16 in range(16):
                ws = wvec[e16]
                e = g * 16 + e16
                for j in range(8):
                    sl = pl.ds(j * 16, 16)
                    rows_v[e, sl] = rows_v[e, sl] * ws
            return c2
        lax.fori_loop(0, _CHUNK // 16, _group, 0)
        pltpu.sync_copy(rows_v, acc.at[ibuf.at[1]], add=True)
        return carry
    lax.fori_loop(0, n_chunks, _chunk, 0)

    plsc.subcore_barrier()

    @pl.when(sid < n_own)
    def _():
        pltpu.sync_copy(acc.at[pl.ds(base, rows_per_owner)],
                        out_hbm.at[cid, pl.ds(base, rows_per_owner)])


def _sc_scatter(xq, edges, w, n_nodes, n_chunks):
    mesh = plsc.VectorSubcoreMesh(core_axis_name="c", subcore_axis_name="s",
                                  num_cores=_NC, num_subcores=_NS)
    kern = pl.kernel(
        functools.partial(_sc_scatter_body, n_nodes, n_chunks),
        out_type=jax.ShapeDtypeStruct((_NC, n_nodes, 128), jnp.float32),
        mesh=mesh,
        scratch_types=[
            pltpu.VMEM((2, _CHUNK), jnp.int32),
            pltpu.VMEM((_CHUNK,), jnp.float32),
            pltpu.VMEM((_CHUNK, 128), jnp.float32),
            pltpu.VMEM_SHARED((n_nodes, 128), jnp.float32),
            pltpu.SemaphoreType.DMA,
        ],
    )
    return kern(xq, edges, w)


def kernel(edge_index, edge_weight, feats, GCN_init_weights,
           update_W, update_U, update_b,
           reset_W, reset_U, reset_b,
           htilda_W, htilda_U, htilda_b,
           dec_W1, dec_b1, dec_W2, dec_b2, pred_flag=1):
    T, n, d = feats.shape
    e = edge_weight.shape[-1]
    n_chunks = e // (_NC * _NS * _CHUNK)
    assert e == _NC * _NS * n_chunks * _CHUNK and n % 1000 == 0 and d == 128

    feats2 = feats[T - 1]
    ei = edge_index[T - 1].astype(jnp.int32)
    shp = (_NC, _NS, n_chunks, _CHUNK)
    dst = ei[0].reshape(shp)
    src = ei[1].reshape(shp)
    edges = jnp.stack([src, dst], axis=3)
    w = edge_weight[T - 1].reshape(shp)

    blk = 1000
    full = pl.BlockSpec((128, 128), lambda i: (0, 0))
    xq = pl.pallas_call(
        _evolve_matmul_body,
        grid=(n // blk,),
        in_specs=[pl.BlockSpec((blk, 128), lambda i: (i, 0))] + [full] * 10,
        out_specs=pl.BlockSpec((blk, 128), lambda i: (i, 0)),
        out_shape=jax.ShapeDtypeStruct((n, 128), jnp.float32),
        scratch_shapes=[pltpu.VMEM((128, 128), jnp.float32)],
    )(feats2, GCN_init_weights, update_W, update_U, update_b,
      reset_W, reset_U, reset_b, htilda_W, htilda_U, htilda_b)

    partial = _sc_scatter(xq, edges, w, n, n_chunks)

    out = pl.pallas_call(
        _decoder_body,
        grid=(n // blk,),
        in_specs=[pl.BlockSpec((_NC, blk, 128), lambda i: (0, i, 0)),
                  pl.BlockSpec((128, 128), lambda i: (0, 0)),
                  pl.BlockSpec((1, 128), lambda i: (0, 0)),
                  pl.BlockSpec((128, 64), lambda i: (0, 0)),
                  pl.BlockSpec((1, 64), lambda i: (0, 0))],
        out_specs=pl.BlockSpec((blk, 64), lambda i: (i, 0)),
        out_shape=jax.ShapeDtypeStruct((n, 64), jnp.float32),
    )(partial, dec_W1, dec_b1.reshape(1, -1), dec_W2, dec_b2.reshape(1, -1))
    return out


# speedup vs baseline: 6.7402x; 1.6138x over previous
import functools

import jax
import jax.numpy as jnp
from jax import lax
from jax.experimental import pallas as pl
from jax.experimental.pallas import tpu as pltpu
from jax.experimental.pallas import tpu_sc as plsc

_NC = 2
_NS = 16
_CHUNK = 80


def _evolve_matmul_body(f_ref, q_ref, uW, uU, ub, rW, rU, rb, hW, hU, hb,
                        o_ref, q3_ref):
    @pl.when(pl.program_id(0) == 0)
    def _():
        hp = jax.lax.Precision.HIGHEST
        dot = functools.partial(jnp.dot, precision=hp,
                                preferred_element_type=jnp.float32)
        Q = q_ref[...]
        for _ in range(3):
            z = Q
            upd = jax.nn.sigmoid(dot(uW[...], z) + dot(uU[...], Q) + ub[...])
            rst = jax.nn.sigmoid(dot(rW[...], z) + dot(rU[...], Q) + rb[...])
            hcap = jnp.tanh(dot(hW[...], z) + dot(hU[...], rst * Q) + hb[...])
            Q = (1.0 - upd) * Q + upd * hcap
        q3_ref[...] = Q

    o_ref[...] = jnp.dot(f_ref[...], q3_ref[...],
                         preferred_element_type=jnp.float32)


def _decoder_body(p_ref, w1, b1, w2, b2, o_ref):
    emb = jnp.maximum(p_ref[0] + p_ref[1], 0.0)
    h = jnp.maximum(
        jnp.dot(emb, w1[...], preferred_element_type=jnp.float32) + b1[...],
        0.0)
    o_ref[...] = jnp.dot(h, w2[...], preferred_element_type=jnp.float32) \
        + b2[...]


def _sc_scatter_body(n_nodes, n_chunks,
                     xq_hbm, edges_hbm, w_hbm, out_hbm,
                     ibuf0, ibuf1, wbuf0, wbuf1, rows0, rows1, acc,
                     si0, si1, sw0, sw1, sr0, sr1):
    cid = lax.axis_index("c")
    sid = lax.axis_index("s")
    ibuf = (ibuf0, ibuf1)
    wbuf = (wbuf0, wbuf1)
    rows = (rows0, rows1)
    si = (si0, si1)
    sw = (sw0, sw1)
    sr = (sr0, sr1)
    n_own = 10
    rows_per_owner = n_nodes // n_own
    base = sid * rows_per_owner

    def _stage(k, b):
        pltpu.async_copy(edges_hbm.at[cid, sid, k], ibuf[b], si[b])
        pltpu.async_copy(w_hbm.at[cid, sid, k], wbuf[b], sw[b])

    def _scale(b):
        def _group(g, c2):
            wvec = wbuf[b][0, pl.ds(g * 16, 16)]
            for e16 in range(16):
                ws = wvec[e16]
                e = g * 16 + e16
                for j in range(8):
                    sl = pl.ds(j * 16, 16)
                    rows[b][e, sl] = rows[b][e, sl] * ws
            return c2
        lax.fori_loop(0, _CHUNK // 16, _group, 0)

    def _consume(k_prev, b):
        pltpu.make_async_copy(xq_hbm.at[ibuf[b].at[0]], rows[b],
                              sr[b]).wait()
        pltpu.make_async_copy(w_hbm.at[cid, sid, k_prev], wbuf[b],
                              sw[b]).wait()
        _scale(b)
        pltpu.sync_copy(rows[b], acc.at[ibuf[b].at[1]], add=True)

    _stage(0, 0)

    @pl.when(sid < n_own)
    def _():
        def _zrow(r, carry):
            for j in range(8):
                rows1[r, pl.ds(j * 16, 16)] = jnp.zeros((16,), jnp.float32)
            return carry
        lax.fori_loop(0, _CHUNK, _zrow, 0)
        nfull, rem = divmod(rows_per_owner, _CHUNK)
        for t in range(nfull):
            pltpu.sync_copy(rows1, acc.at[pl.ds(base + t * _CHUNK, _CHUNK)])
        if rem:
            pltpu.sync_copy(rows1.at[pl.ds(0, rem)],
                            acc.at[pl.ds(base + nfull * _CHUNK, rem)])

    pltpu.make_async_copy(edges_hbm.at[cid, sid, 0], ibuf0, si0).wait()
    pltpu.async_copy(xq_hbm.at[ibuf0.at[0]], rows0, sr0)
    _stage(1, 1)
    plsc.subcore_barrier()

    def _half(k, b):
        pltpu.make_async_copy(edges_hbm.at[cid, sid, k], ibuf[b],
                              si[b]).wait()
        pltpu.async_copy(xq_hbm.at[ibuf[b].at[0]], rows[b], sr[b])
        _consume(k - 1, 1 - b)

        @pl.when(k < n_chunks - 1)
        def _():
            _stage(k + 1, 1 - b)

    def _pair(i, carry):
        _half(2 * i + 1, 1)
        _half(2 * i + 2, 0)
        return carry
    lax.fori_loop(0, (n_chunks - 1) // 2, _pair, 0)
    _consume(n_chunks - 1, (n_chunks - 1) % 2)

    plsc.subcore_barrier()

    @pl.when(sid < n_own)
    def _():
        pltpu.sync_copy(acc.at[pl.ds(base, rows_per_owner)],
                        out_hbm.at[cid, pl.ds(base, rows_per_owner)])


def _sc_scatter(xq, edges, w, n_nodes, n_chunks):
    mesh = plsc.VectorSubcoreMesh(core_axis_name="c", subcore_axis_name="s",
                                  num_cores=_NC, num_subcores=_NS)
    kern = pl.kernel(
        functools.partial(_sc_scatter_body, n_nodes, n_chunks),
        out_type=jax.ShapeDtypeStruct((_NC, n_nodes, 128), jnp.float32),
        mesh=mesh,
        scratch_types=[
            pltpu.VMEM((2, _CHUNK), jnp.int32),
            pltpu.VMEM((2, _CHUNK), jnp.int32),
            pltpu.VMEM((1, _CHUNK), jnp.float32),
            pltpu.VMEM((1, _CHUNK), jnp.float32),
            pltpu.VMEM((_CHUNK, 128), jnp.float32),
            pltpu.VMEM((_CHUNK, 128), jnp.float32),
            pltpu.VMEM_SHARED((n_nodes, 128), jnp.float32),
            pltpu.SemaphoreType.DMA,
            pltpu.SemaphoreType.DMA,
            pltpu.SemaphoreType.DMA,
            pltpu.SemaphoreType.DMA,
            pltpu.SemaphoreType.DMA,
            pltpu.SemaphoreType.DMA,
        ],
    )
    return kern(xq, edges, w)


def kernel(edge_index, edge_weight, feats, GCN_init_weights,
           update_W, update_U, update_b,
           reset_W, reset_U, reset_b,
           htilda_W, htilda_U, htilda_b,
           dec_W1, dec_b1, dec_W2, dec_b2, pred_flag=1):
    T, n, d = feats.shape
    e = edge_weight.shape[-1]
    n_chunks = e // (_NC * _NS * _CHUNK)
    assert e == _NC * _NS * n_chunks * _CHUNK and n % 1000 == 0 and d == 128
    assert n_chunks % 2 == 1

    feats2 = feats[T - 1]
    ei = edge_index[T - 1].astype(jnp.int32)
    shp = (_NC, _NS, n_chunks, _CHUNK)
    dst = ei[0].reshape(shp)
    src = ei[1].reshape(shp)
    edges = jnp.stack([src, dst], axis=3)
    w = edge_weight[T - 1].reshape(_NC, _NS, n_chunks, 1, _CHUNK)

    blk = 1000
    full = pl.BlockSpec((128, 128), lambda i: (0, 0))
    xq = pl.pallas_call(
        _evolve_matmul_body,
        grid=(n // blk,),
        in_specs=[pl.BlockSpec((blk, 128), lambda i: (i, 0))] + [full] * 10,
        out_specs=pl.BlockSpec((blk, 128), lambda i: (i, 0)),
        out_shape=jax.ShapeDtypeStruct((n, 128), jnp.float32),
        scratch_shapes=[pltpu.VMEM((128, 128), jnp.float32)],
    )(feats2, GCN_init_weights, update_W, update_U, update_b,
      reset_W, reset_U, reset_b, htilda_W, htilda_U, htilda_b)

    partial = _sc_scatter(xq, edges, w, n, n_chunks)

    out = pl.pallas_call(
        _decoder_body,
        grid=(n // blk,),
        in_specs=[pl.BlockSpec((_NC, blk, 128), lambda i: (0, i, 0)),
                  pl.BlockSpec((128, 128), lambda i: (0, 0)),
                  pl.BlockSpec((1, 128), lambda i: (0, 0)),
                  pl.BlockSpec((128, 64), lambda i: (0, 0)),
                  pl.BlockSpec((1, 64), lambda i: (0, 0))],
        out_specs=pl.BlockSpec((blk, 64), lambda i: (i, 0)),
        out_shape=jax.ShapeDtypeStruct((n, 64), jnp.float32),
    )(partial, dec_W1, dec_b1.reshape(1, -1), dec_W2, dec_b2.reshape(1, -1))
    return out


# speedup vs baseline: 7.9612x; 1.1811x over previous
import functools

import jax
import jax.numpy as jnp
from jax import lax
from jax.experimental import pallas as pl
from jax.experimental.pallas import tpu as pltpu
from jax.experimental.pallas import tpu_sc as plsc

_NC = 2
_NS = 16
_CHUNK = 80


def _evolve_matmul_body(f_ref, q_ref, uW, uU, ub, rW, rU, rb, hW, hU, hb,
                        o_ref, q3_ref):
    @pl.when(pl.program_id(0) == 0)
    def _():
        hp = jax.lax.Precision.HIGHEST
        dot = functools.partial(jnp.dot, precision=hp,
                                preferred_element_type=jnp.float32)
        Q = q_ref[...]
        for _ in range(3):
            z = Q
            upd = jax.nn.sigmoid(dot(uW[...], z) + dot(uU[...], Q) + ub[...])
            rst = jax.nn.sigmoid(dot(rW[...], z) + dot(rU[...], Q) + rb[...])
            hcap = jnp.tanh(dot(hW[...], z) + dot(hU[...], rst * Q) + hb[...])
            Q = (1.0 - upd) * Q + upd * hcap
        q3_ref[...] = Q

    o_ref[...] = jnp.dot(f_ref[0], q3_ref[...],
                         preferred_element_type=jnp.float32)


def _decoder_body(p_ref, w1, b1, w2, b2, o_ref):
    emb = jnp.maximum(p_ref[0] + p_ref[1], 0.0)
    h = jnp.maximum(
        jnp.dot(emb, w1[...], preferred_element_type=jnp.float32) + b1[...],
        0.0)
    o_ref[...] = jnp.dot(h, w2[...], preferred_element_type=jnp.float32) \
        + b2[...]


def _sc_scatter_body(n_nodes, n_chunks,
                     xq_hbm, src_hbm, dst_hbm, w_hbm, out_hbm,
                     ibs0, ibs1, ibd0, ibd1, sbuf0, sbuf1,
                     wbuf0, wbuf1, rows0, rows1, acc,
                     sis0, sis1, sid0, sid1, sw0, sw1, sr0, sr1, ss0, ss1):
    cid = lax.axis_index("c")
    sid_ = lax.axis_index("s")
    ibs = (ibs0, ibs1)
    ibd = (ibd0, ibd1)
    sbuf = (sbuf0, sbuf1)
    wbuf = (wbuf0, wbuf1)
    rows = (rows0, rows1)
    sis = (sis0, sis1)
    sid = (sid0, sid1)
    sw = (sw0, sw1)
    sr = (sr0, sr1)
    ss = (ss0, ss1)
    n_own = 10
    rows_per_owner = n_nodes // n_own
    base = sid_ * rows_per_owner

    def _stage(k, b):
        pltpu.async_copy(src_hbm.at[cid, sid_, k], ibs[b], sis[b])
        pltpu.async_copy(dst_hbm.at[cid, sid_, k], ibd[b], sid[b])
        pltpu.async_copy(w_hbm.at[cid, sid_, k], wbuf[b], sw[b])

    def _scale(b):
        def _group(g, c2):
            wvec = wbuf[b][0, pl.ds(g * 16, 16)]
            for e16 in range(16):
                ws = wvec[e16]
                e = g * 16 + e16
                for j in range(8):
                    sl = pl.ds(j * 16, 16)
                    rows[b][e, sl] = rows[b][e, sl] * ws
            return c2
        lax.fori_loop(0, _CHUNK // 16, _group, 0)

    def _consume(k_prev, b):
        pltpu.make_async_copy(xq_hbm.at[ibs[b].at[0]], rows[b],
                              sr[b]).wait()
        pltpu.make_async_copy(w_hbm.at[cid, sid_, k_prev], wbuf[b],
                              sw[b]).wait()
        _scale(b)
        pltpu.make_async_copy(dst_hbm.at[cid, sid_, k_prev], ibd[b],
                              sid[b]).wait()
        for g5 in range(_CHUNK // 16):
            sl = pl.ds(g5 * 16, 16)
            sbuf[b][0, sl] = ibd[b][0, sl]
        pltpu.async_copy(rows[b], acc.at[sbuf[b].at[0]], ss[b], add=True)

    def _drain_scatter(b):
        pltpu.make_async_copy(rows[b], acc.at[sbuf[b].at[0]], ss[b]).wait()

    _stage(0, 0)

    @pl.when(sid_ < n_own)
    def _():
        def _zrow(r, carry):
            for j in range(8):
                rows1[r, pl.ds(j * 16, 16)] = jnp.zeros((16,), jnp.float32)
            return carry
        lax.fori_loop(0, _CHUNK, _zrow, 0)
        nfull, rem = divmod(rows_per_owner, _CHUNK)
        for t in range(nfull):
            pltpu.sync_copy(rows1, acc.at[pl.ds(base + t * _CHUNK, _CHUNK)])
        if rem:
            pltpu.sync_copy(rows1.at[pl.ds(0, rem)],
                            acc.at[pl.ds(base + nfull * _CHUNK, rem)])

    pltpu.make_async_copy(src_hbm.at[cid, sid_, 0], ibs0, sis0).wait()
    pltpu.async_copy(xq_hbm.at[ibs0.at[0]], rows0, sr0)
    _stage(1, 1)
    plsc.subcore_barrier()

    def _half(k, b):
        @pl.when(k > 1)
        def _():
            _drain_scatter(b)
        pltpu.make_async_copy(src_hbm.at[cid, sid_, k], ibs[b],
                              sis[b]).wait()
        pltpu.async_copy(xq_hbm.at[ibs[b].at[0]], rows[b], sr[b])
        _consume(k - 1, 1 - b)

        @pl.when(k < n_chunks - 1)
        def _():
            _stage(k + 1, 1 - b)

    def _pair(i, carry):
        _half(2 * i + 1, 1)
        _half(2 * i + 2, 0)
        return carry
    lax.fori_loop(0, (n_chunks - 1) // 2, _pair, 0)
    last = n_chunks - 1
    _consume(last, last % 2)
    _drain_scatter(last % 2)
    _drain_scatter(1 - last % 2)

    plsc.subcore_barrier()

    @pl.when(sid_ < n_own)
    def _():
        pltpu.sync_copy(acc.at[pl.ds(base, rows_per_owner)],
                        out_hbm.at[cid, pl.ds(base, rows_per_owner)])


def _sc_scatter(xq, src, dst, w, n_nodes, n_chunks):
    mesh = plsc.VectorSubcoreMesh(core_axis_name="c", subcore_axis_name="s",
                                  num_cores=_NC, num_subcores=_NS)
    kern = pl.kernel(
        functools.partial(_sc_scatter_body, n_nodes, n_chunks),
        out_type=jax.ShapeDtypeStruct((_NC, n_nodes, 128), jnp.float32),
        mesh=mesh,
        scratch_types=(
            [pltpu.VMEM((1, _CHUNK), jnp.int32)] * 6
            + [pltpu.VMEM((1, _CHUNK), jnp.float32)] * 2
            + [pltpu.VMEM((_CHUNK, 128), jnp.float32)] * 2
            + [pltpu.VMEM_SHARED((n_nodes, 128), jnp.float32)]
            + [pltpu.SemaphoreType.DMA] * 10
        ),
    )
    return kern(xq, src, dst, w)


def kernel(edge_index, edge_weight, feats, GCN_init_weights,
           update_W, update_U, update_b,
           reset_W, reset_U, reset_b,
           htilda_W, htilda_U, htilda_b,
           dec_W1, dec_b1, dec_W2, dec_b2, pred_flag=1):
    T, n, d = feats.shape
    e = edge_weight.shape[-1]
    n_chunks = e // (_NC * _NS * _CHUNK)
    assert e == _NC * _NS * n_chunks * _CHUNK and n % 1000 == 0 and d == 128
    assert n_chunks % 2 == 1

    ei = edge_index[T - 1].astype(jnp.int32)
    shp = (_NC, _NS, n_chunks, 1, _CHUNK)
    dst = ei[0].reshape(shp)
    src = ei[1].reshape(shp)
    w = edge_weight[T - 1].reshape(shp)

    blk = 1000
    full = pl.BlockSpec((128, 128), lambda i: (0, 0))
    xq = pl.pallas_call(
        _evolve_matmul_body,
        grid=(n // blk,),
        in_specs=[pl.BlockSpec((1, blk, 128), lambda i: (T - 1, i, 0))]
        + [full] * 10,
        out_specs=pl.BlockSpec((blk, 128), lambda i: (i, 0)),
        out_shape=jax.ShapeDtypeStruct((n, 128), jnp.float32),
        scratch_shapes=[pltpu.VMEM((128, 128), jnp.float32)],
    )(feats, GCN_init_weights, update_W, update_U, update_b,
      reset_W, reset_U, reset_b, htilda_W, htilda_U, htilda_b)

    partial = _sc_scatter(xq, src, dst, w, n, n_chunks)

    out = pl.pallas_call(
        _decoder_body,
        grid=(n // blk,),
        in_specs=[pl.BlockSpec((_NC, blk, 128), lambda i: (0, i, 0)),
                  pl.BlockSpec((128, 128), lambda i: (0, 0)),
                  pl.BlockSpec((1, 128), lambda i: (0, 0)),
                  pl.BlockSpec((128, 64), lambda i: (0, 0)),
                  pl.BlockSpec((1, 64), lambda i: (0, 0))],
        out_specs=pl.BlockSpec((blk, 64), lambda i: (i, 0)),
        out_shape=jax.ShapeDtypeStruct((n, 64), jnp.float32),
    )(partial, dec_W1, dec_b1.reshape(1, -1), dec_W2, dec_b2.reshape(1, -1))
    return out
